# Initial kernel scaffold; baseline (speedup 1.0000x reference)
import jax
import jax.numpy as jnp
from jax.experimental import pallas as pl


def kernel(x, W1, b1, W2, b2):
    raise NotImplementedError("write your pallas kernel here")


# speedup vs baseline: 116.6076x; 116.6076x over previous
import functools

import jax
import jax.numpy as jnp
from jax import lax
from jax.experimental import pallas as pl

_B, _S, _F = 8, 128, 2048
_HID = 64
_K = 8


def _gcn_kernel(x_ref, w1_ref, b1_ref, w2_ref, b2_ref, out_ref):
    xb = x_ref[0]
    f = _F
    kk = min(_K, f - 1)

    v = jnp.mean(xb, axis=0, keepdims=True)
    norm = jnp.sqrt(jnp.sum(v * v))
    vn = v / jnp.maximum(norm, 1e-12)

    iota = lax.broadcasted_iota(jnp.int32, (1, f), 1)

    def extract(vals, largest):
        work = vals
        mask = jnp.zeros((1, f), jnp.float32)
        for r in range(kk + 1):
            m = jnp.max(work) if largest else jnp.min(work)
            first = jnp.min(jnp.where(work == m, iota, f))
            onehot = iota == first
            if r > 0:
                mask = jnp.where(onehot, 1.0, mask)
            work = jnp.where(onehot, -jnp.inf if largest else jnp.inf, work)
        return mask

    p = extract(vn, True)
    n = extract(vn, False)
    a = (vn > 0).astype(jnp.float32)
    b = (vn < 0).astype(jnp.float32)
    ap, an = a * p, a * n
    bp, bn = b * p, b * n
    m = ap + bn

    sum_masks = jnp.concatenate([p, n, a, b, ap, bp, an, bn], axis=0)
    row_masks = jnp.concatenate([a, b, p, n, ap, an, bp, bn], axis=0)
    counts = jnp.sum(sum_masks, axis=1, keepdims=True)
    sgn = jnp.where(lax.broadcasted_iota(jnp.int32, (8, 1), 0) < 4, 1.0, -1.0)

    deg = (1.0 - m
           + jnp.sum(sgn * counts * row_masks, axis=0, keepdims=True))
    inv_deg = 1.0 / deg

    def apply_adj(h):
        s = lax.dot_general(h, sum_masks, (((1,), (1,)), ((), ())),
                            preferred_element_type=jnp.float32)
        coef = s * sgn[:, 0][None, :]
        corr = lax.dot_general(coef, row_masks, (((1,), (0,)), ((), ())),
                               preferred_element_type=jnp.float32)
        return (h * (1.0 - m) + corr) * inv_deg

    h1 = lax.dot_general(w1_ref[...], xb, (((0,), (0,)), ((), ())),
                         preferred_element_type=jnp.float32)
    h1 = h1 + b1_ref[...]
    g1 = jnp.maximum(apply_adj(h1), 0.0)

    h2 = lax.dot_general(w2_ref[...], g1, (((0,), (0,)), ((), ())),
                         preferred_element_type=jnp.float32)
    h2 = h2 + b2_ref[...]
    out_ref[0] = apply_adj(h2)


@functools.partial(jax.jit, static_argnums=())
def kernel(x, W1, b1, W2, b2):
    b1c = b1.reshape(_HID, 1)
    b2c = b2.reshape(_S, 1)
    return pl.pallas_call(
        _gcn_kernel,
        grid=(_B,),
        in_specs=[
            pl.BlockSpec((1, _S, _F), lambda i: (i, 0, 0)),
            pl.BlockSpec((_S, _HID), lambda i: (0, 0)),
            pl.BlockSpec((_HID, 1), lambda i: (0, 0)),
            pl.BlockSpec((_HID, _S), lambda i: (0, 0)),
            pl.BlockSpec((_S, 1), lambda i: (0, 0)),
        ],
        out_specs=pl.BlockSpec((1, _S, _F), lambda i: (i, 0, 0)),
        out_shape=jax.ShapeDtypeStruct((_B, _S, _F), jnp.float32),
    )(x, W1, b1c, W2, b2c)


# speedup vs baseline: 269.2872x; 2.3093x over previous
import functools

import jax
import jax.numpy as jnp
from jax import lax
from jax.experimental import pallas as pl
from jax.experimental.pallas import tpu as pltpu

_B, _S, _F = 8, 128, 2048
_HID = 64
_K = 8


def _stage_a_kernel(x_ref, w1_ref, b1_ref, h1_ref, v_ref):
    xb = x_ref[0]
    v_ref[0] = jnp.mean(xb, axis=0, keepdims=True)
    h1 = lax.dot_general(w1_ref[...], xb, (((0,), (0,)), ((), ())),
                         preferred_element_type=jnp.float32)
    h1_ref[0] = h1 + b1_ref[...]


def _stage_c_kernel(h1_ref, v_ref, w2_ref, b2_ref, out_ref,
                    p_scr, n_scr, a_scr, b_scr, w_scr):
    f = _F
    kk = min(_K, f - 1)
    pid = pl.program_id(0)

    @pl.when(pid == 0)
    def _compute_masks():
        v = v_ref[...].reshape(_B, f)
        norm = jnp.sqrt(jnp.sum(v * v, axis=1, keepdims=True))
        vn = v / jnp.maximum(norm, 1e-12)

        def extract(vals, largest):
            work = vals
            mask = jnp.zeros((_B, f), jnp.float32)
            for r in range(kk + 1):
                m = (jnp.max(work, axis=1, keepdims=True) if largest
                     else jnp.min(work, axis=1, keepdims=True))
                sel = work == m
                if r > 0:
                    mask = jnp.where(sel, 1.0, mask)
                work = jnp.where(sel, -jnp.inf if largest else jnp.inf, work)
            return mask

        p = extract(vn, True)
        n = extract(vn, False)
        a = (vn > 0).astype(jnp.float32)
        b = (vn < 0).astype(jnp.float32)
        ap, an = a * p, a * n
        bp, bn = b * p, b * n
        m = ap + bn

        def rsum(t):
            return jnp.sum(t, axis=1, keepdims=True)

        deg = (1.0 - m
               + rsum(p) * a + rsum(n) * b + rsum(a) * p + rsum(b) * n
               - rsum(ap) * ap - rsum(bp) * an - rsum(an) * bp - rsum(bn) * bn)
        p_scr[...] = p
        n_scr[...] = n
        a_scr[...] = a
        b_scr[...] = b
        w_scr[...] = 1.0 / deg

    bsl = pl.ds(pid, 1)
    p = p_scr[bsl, :]
    n = n_scr[bsl, :]
    a = a_scr[bsl, :]
    b = b_scr[bsl, :]
    inv_deg = w_scr[bsl, :]
    ap, an = a * p, a * n
    bp, bn = b * p, b * n
    w = (1.0 - (ap + bn)) * inv_deg
    sum_masks = jnp.concatenate([p, n, a, b, ap, bp, an, bn], axis=0)
    row_masks = jnp.concatenate([a, b, p, n, ap, an, bp, bn], axis=0)
    sgn = jnp.where(lax.broadcasted_iota(jnp.int32, (1, 8), 1) < 4, 1.0, -1.0)

    def apply_adj(h):
        s = lax.dot_general(h, sum_masks, (((1,), (1,)), ((), ())),
                            preferred_element_type=jnp.float32)
        corr = lax.dot_general(s * sgn, row_masks, (((1,), (0,)), ((), ())),
                               preferred_element_type=jnp.float32)
        return h * w + corr * inv_deg

    g1 = jnp.maximum(apply_adj(h1_ref[0]), 0.0)
    h2 = lax.dot_general(w2_ref[...], g1, (((0,), (0,)), ((), ())),
                         preferred_element_type=jnp.float32)
    h2 = h2 + b2_ref[...]
    out_ref[0] = apply_adj(h2)


@functools.partial(jax.jit, static_argnums=())
def kernel(x, W1, b1, W2, b2):
    b1c = b1.reshape(_HID, 1)
    b2c = b2.reshape(_S, 1)
    h1, v = pl.pallas_call(
        _stage_a_kernel,
        grid=(_B,),
        in_specs=[
            pl.BlockSpec((1, _S, _F), lambda i: (i, 0, 0)),
            pl.BlockSpec((_S, _HID), lambda i: (0, 0)),
            pl.BlockSpec((_HID, 1), lambda i: (0, 0)),
        ],
        out_specs=[
            pl.BlockSpec((1, _HID, _F), lambda i: (i, 0, 0)),
            pl.BlockSpec((1, 1, _F), lambda i: (i, 0, 0)),
        ],
        out_shape=[
            jax.ShapeDtypeStruct((_B, _HID, _F), jnp.float32),
            jax.ShapeDtypeStruct((_B, 1, _F), jnp.float32),
        ],
    )(x, W1, b1c)
    return pl.pallas_call(
        _stage_c_kernel,
        grid=(_B,),
        in_specs=[
            pl.BlockSpec((1, _HID, _F), lambda i: (i, 0, 0)),
            pl.BlockSpec((_B, 1, _F), lambda i: (0, 0, 0)),
            pl.BlockSpec((_HID, _S), lambda i: (0, 0)),
            pl.BlockSpec((_S, 1), lambda i: (0, 0)),
        ],
        out_specs=pl.BlockSpec((1, _S, _F), lambda i: (i, 0, 0)),
        out_shape=jax.ShapeDtypeStruct((_B, _S, _F), jnp.float32),
        scratch_shapes=[pltpu.VMEM((_B, _F), jnp.float32)] * 5,
    )(h1, v, W2, b2c)


# speedup vs baseline: 269.6699x; 1.0014x over previous
import functools

import jax
import jax.numpy as jnp
from jax import lax
from jax.experimental import pallas as pl
from jax.experimental.pallas import tpu as pltpu

_B, _S, _F = 8, 128, 2048
_HID = 64
_K = 8


def _stage_a_kernel(x_ref, w1_ref, h1_ref, v_ref):
    xb = x_ref[0]
    v_ref[0] = jnp.mean(xb, axis=0, keepdims=True)
    h1_ref[0] = lax.dot_general(w1_ref[...], xb, (((0,), (0,)), ((), ())),
                                preferred_element_type=jnp.float32)


def _stage_c_kernel(h1_ref, v_ref, b1_ref, w2_ref, b2_ref, out_ref,
                    sm_scr, rm_scr, cnt_scr):
    f = _F
    kk = min(_K, f - 1)
    pid = pl.program_id(0)

    @pl.when(pid == 0)
    def _compute_masks():
        v = v_ref[...].reshape(_B, f)
        norm = jnp.sqrt(jnp.sum(v * v, axis=1, keepdims=True))
        vn = v / jnp.maximum(norm, 1e-12)

        work = jnp.concatenate([vn, -vn], axis=0)
        mask = jnp.zeros((2 * _B, f), jnp.float32)
        for r in range(kk + 1):
            mx = jnp.max(work, axis=1, keepdims=True)
            sel = work == mx
            if r > 0:
                mask = jnp.where(sel, 1.0, mask)
            work = jnp.where(sel, -jnp.inf, work)
        p = mask[:_B]
        n = mask[_B:]
        a = (vn > 0).astype(jnp.float32)
        b = (vn < 0).astype(jnp.float32)
        ap, an = a * p, a * n
        bp, bn = b * p, b * n
        m = ap + bn

        def rsum(t):
            return jnp.sum(t, axis=1, keepdims=True)

        sum_rows = (p, n, a, b, ap, bp, an, bn)
        row_rows = (a, b, p, n, ap, an, bp, bn)
        cnts = [rsum(t) for t in sum_rows]
        deg = (1.0 - m
               + cnts[0] * a + cnts[1] * b + cnts[2] * p + cnts[3] * n
               - cnts[4] * ap - cnts[5] * an - cnts[6] * bp - cnts[7] * bn)
        inv_deg = 1.0 / deg
        for k in range(8):
            sm_scr[:, k, :] = sum_rows[k]
            rm_scr[:, k, :] = row_rows[k] * inv_deg
        rm_scr[:, 8, :] = (1.0 - m) * inv_deg
        cnt_scr[...] = jnp.concatenate(cnts, axis=1)

    sm = sm_scr[pid]
    rm = rm_scr[pid]
    w = rm[8:9]
    cnt = cnt_scr[pid]
    sgn = jnp.where(lax.broadcasted_iota(jnp.int32, (1, 8), 1) < 4, 1.0, -1.0)

    def layer(h_raw, bias):
        s = lax.dot_general(h_raw, sm, (((1,), (1,)), ((), ())),
                            preferred_element_type=jnp.float32)
        coef = jnp.concatenate([(s + bias * cnt[None, :]) * sgn, bias], axis=1)
        corr = lax.dot_general(coef, rm, (((1,), (0,)), ((), ())),
                               preferred_element_type=jnp.float32)
        return h_raw * w + corr

    g1 = jnp.maximum(layer(h1_ref[0], b1_ref[...]), 0.0)
    h2 = lax.dot_general(w2_ref[...], g1, (((0,), (0,)), ((), ())),
                         preferred_element_type=jnp.float32)
    out_ref[0] = layer(h2, b2_ref[...])


@functools.partial(jax.jit, static_argnums=())
def kernel(x, W1, b1, W2, b2):
    b1c = b1.reshape(_HID, 1)
    b2c = b2.reshape(_S, 1)
    h1, v = pl.pallas_call(
        _stage_a_kernel,
        grid=(_B,),
        in_specs=[
            pl.BlockSpec((1, _S, _F), lambda i: (i, 0, 0)),
            pl.BlockSpec((_S, _HID), lambda i: (0, 0)),
        ],
        out_specs=[
            pl.BlockSpec((1, _HID, _F), lambda i: (i, 0, 0)),
            pl.BlockSpec((1, 1, _F), lambda i: (i, 0, 0)),
        ],
        out_shape=[
            jax.ShapeDtypeStruct((_B, _HID, _F), jnp.float32),
            jax.ShapeDtypeStruct((_B, 1, _F), jnp.float32),
        ],
    )(x, W1)
    return pl.pallas_call(
        _stage_c_kernel,
        grid=(_B,),
        in_specs=[
            pl.BlockSpec((1, _HID, _F), lambda i: (i, 0, 0)),
            pl.BlockSpec((_B, 1, _F), lambda i: (0, 0, 0)),
            pl.BlockSpec((_HID, 1), lambda i: (0, 0)),
            pl.BlockSpec((_HID, _S), lambda i: (0, 0)),
            pl.BlockSpec((_S, 1), lambda i: (0, 0)),
        ],
        out_specs=pl.BlockSpec((1, _S, _F), lambda i: (i, 0, 0)),
        out_shape=jax.ShapeDtypeStruct((_B, _S, _F), jnp.float32),
        scratch_shapes=[
            pltpu.VMEM((_B, 8, _F), jnp.float32),
            pltpu.VMEM((_B, 9, _F), jnp.float32),
            pltpu.VMEM((_B, 8), jnp.float32),
        ],
    )(h1, v, b1c, W2, b2c)


# speedup vs baseline: 311.3532x; 1.1546x over previous
import functools

import jax
import jax.numpy as jnp
from jax import lax
from jax.experimental import pallas as pl
from jax.experimental.pallas import tpu as pltpu

_B, _S, _F = 8, 128, 2048
_HID = 64
_K = 8


def _fused_kernel(x_ref, w1_ref, b1_ref, w2_ref, b2_ref, out_ref,
                  h1_scr, v_scr, sm_scr, rm_scr, cnt_scr, obuf, sems):
    f = _F
    kk = min(_K, f - 1)
    i = pl.program_id(0)

    @pl.when(i < _B)
    def _stage_a():
        xb = x_ref[0]
        v_scr[pl.ds(i, 1), :] = jnp.mean(xb, axis=0, keepdims=True)
        h1_scr[pl.ds(i, 1)] = lax.dot_general(
            w1_ref[...], xb, (((0,), (0,)), ((), ())),
            preferred_element_type=jnp.float32)[None]

    @pl.when(i == _B)
    def _compute_masks():
        v = v_scr[...]
        norm = jnp.sqrt(jnp.sum(v * v, axis=1, keepdims=True))
        vn = v / jnp.maximum(norm, 1e-12)

        work = jnp.concatenate([vn, -vn], axis=0)
        mask = jnp.zeros((2 * _B, f), jnp.float32)
        for r in range(kk + 1):
            mx = jnp.max(work, axis=1, keepdims=True)
            sel = work == mx
            if r > 0:
                mask = jnp.where(sel, 1.0, mask)
            work = jnp.where(sel, -jnp.inf, work)
        p = mask[:_B]
        n = mask[_B:]
        a = (vn > 0).astype(jnp.float32)
        b = (vn < 0).astype(jnp.float32)
        ap, an = a * p, a * n
        bp, bn = b * p, b * n
        m = ap + bn

        def rsum(t):
            return jnp.sum(t, axis=1, keepdims=True)

        sum_rows = (p, n, a, b, ap, bp, an, bn)
        row_rows = (a, b, p, n, ap, an, bp, bn)
        cnts = [rsum(t) for t in sum_rows]
        deg = (1.0 - m
               + cnts[0] * a + cnts[1] * b + cnts[2] * p + cnts[3] * n
               - cnts[4] * ap - cnts[5] * an - cnts[6] * bp - cnts[7] * bn)
        inv_deg = 1.0 / deg
        for k in range(8):
            sm_scr[:, k, :] = sum_rows[k]
            rm_scr[:, k, :] = row_rows[k] * inv_deg
        rm_scr[:, 8, :] = (1.0 - m) * inv_deg
        cnt_scr[...] = jnp.concatenate(cnts, axis=1)

    @pl.when(i >= _B)
    def _stage_c():
        bidx = i - _B
        par = lax.rem(bidx, 2)

        @pl.when(bidx >= 2)
        def _wait_prev():
            pltpu.make_async_copy(
                obuf.at[par], out_ref.at[bidx - 2], sems.at[par]).wait()

        sm = sm_scr[bidx]
        rm = rm_scr[bidx]
        w = rm[8:9]
        cnt = cnt_scr[bidx]
        sgn = jnp.where(
            lax.broadcasted_iota(jnp.int32, (1, 8), 1) < 4, 1.0, -1.0)

        def layer(h_raw, bias):
            s = lax.dot_general(h_raw, sm, (((1,), (1,)), ((), ())),
                                preferred_element_type=jnp.float32)
            coef = jnp.concatenate(
                [(s + bias * cnt[None, :]) * sgn, bias], axis=1)
            corr = lax.dot_general(coef, rm, (((1,), (0,)), ((), ())),
                                   preferred_element_type=jnp.float32)
            return h_raw * w + corr

        g1 = jnp.maximum(layer(h1_scr[bidx], b1_ref[...]), 0.0)
        h2 = lax.dot_general(w2_ref[...], g1, (((0,), (0,)), ((), ())),
                             preferred_element_type=jnp.float32)
        obuf[par] = layer(h2, b2_ref[...])
        cp = pltpu.make_async_copy(obuf.at[par], out_ref.at[bidx],
                                   sems.at[par])
        cp.start()

        @pl.when(bidx == _B - 1)
        def _drain_tail():
            pltpu.make_async_copy(
                obuf.at[1 - par], out_ref.at[bidx - 1], sems.at[1 - par]).wait()
            pltpu.make_async_copy(
                obuf.at[par], out_ref.at[bidx], sems.at[par]).wait()


@functools.partial(jax.jit, static_argnums=())
def kernel(x, W1, b1, W2, b2):
    b1c = b1.reshape(_HID, 1)
    b2c = b2.reshape(_S, 1)
    out = pl.pallas_call(
        _fused_kernel,
        grid=(2 * _B,),
        in_specs=[
            pl.BlockSpec((1, _S, _F),
                         lambda i: (jnp.minimum(i, _B - 1), 0, 0)),
            pl.BlockSpec((_S, _HID), lambda i: (0, 0)),
            pl.BlockSpec((_HID, 1), lambda i: (0, 0)),
            pl.BlockSpec((_HID, _S), lambda i: (0, 0)),
            pl.BlockSpec((_S, 1), lambda i: (0, 0)),
        ],
        out_specs=pl.BlockSpec(memory_space=pl.ANY),
        out_shape=jax.ShapeDtypeStruct((_B, _S, _F), jnp.float32),
        scratch_shapes=[
            pltpu.VMEM((_B, _HID, _F), jnp.float32),
            pltpu.VMEM((_B, _F), jnp.float32),
            pltpu.VMEM((_B, 8, _F), jnp.float32),
            pltpu.VMEM((_B, 9, _F), jnp.float32),
            pltpu.VMEM((_B, 8), jnp.float32),
            pltpu.VMEM((2, _S, _F), jnp.float32),
            pltpu.SemaphoreType.DMA((2,)),
        ],
    )(x, W1, b1c, W2, b2c)
    return out
